# Initial kernel scaffold; baseline (speedup 1.0000x reference)
import functools

import jax
import jax.numpy as jnp
from jax import lax
from jax.experimental import pallas as pl
from jax.experimental.pallas import tpu as pltpu
from jax.experimental.pallas import tpu_sc as plsc

N = 10000
E = 320000
D = 128
NC = 2
NS = 16
NW = NC * NS
CHUNK = 128
NCH = 80
EPW = NCH * CHUNK
EPAD = NW * EPW
NB = 10240
RPT = NB // NS
ORPT = N // NS

_mesh = plsc.VectorSubcoreMesh(core_axis_name="c", subcore_axis_name="s")


@functools.partial(
    pl.kernel,
    out_type=jax.ShapeDtypeStruct((NW, NB), jnp.float32),
    mesh=_mesh,
    scratch_types=[
        pltpu.VMEM((NCH, CHUNK), jnp.int32),
        pltpu.VMEM((NB,), jnp.float32),
    ],
)
def _deg_kernel(dst_hbm, out_hbm, dst_v, hist_v):
    wid = lax.axis_index("c") * NS + lax.axis_index("s")
    pltpu.sync_copy(dst_hbm.at[wid], dst_v)
    zeros16 = jnp.zeros((16,), jnp.float32)

    def zero_body(i, _):
        hist_v[pl.ds(i * 16, 16)] = zeros16
        return _

    lax.fori_loop(0, NB // 16, zero_body, 0)
    ones16 = jnp.ones((16,), jnp.float32)

    def edge_body(j, _):
        for k in range(CHUNK // 16):
            idx = dst_v[j, pl.ds(k * 16, 16)]
            plsc.addupdate_scatter(hist_v, [idx], ones16)
        return _

    lax.fori_loop(0, NCH, edge_body, 0)
    pltpu.sync_copy(hist_v, out_hbm.at[wid])


@functools.partial(
    pl.kernel,
    out_type=jax.ShapeDtypeStruct((NC, N, D), jnp.float32),
    mesh=_mesh,
    scratch_types=[
        pltpu.VMEM((NCH, CHUNK), jnp.int32),
        pltpu.VMEM((NCH, CHUNK), jnp.int32),
        pltpu.VMEM((CHUNK, D), jnp.float32),
        pltpu.VMEM((CHUNK, D), jnp.float32),
        pltpu.VMEM_SHARED((NB, D), jnp.float32),
        pltpu.SemaphoreType.DMA,
    ],
)
def _rowagg_kernel(src_hbm, dst_hbm, g_hbm, out_hbm,
                   src_v, dst_v, rows_v, zbuf_v, acc_sh, gsem):
    cid = lax.axis_index("c")
    sid = lax.axis_index("s")
    wid = cid * NS + sid
    pltpu.sync_copy(src_hbm.at[wid], src_v)
    pltpu.sync_copy(dst_hbm.at[wid], dst_v)

    zeros16 = jnp.zeros((16,), jnp.float32)

    def zbuf_body(i, _):
        zbuf_v[i // 8, pl.ds((i % 8) * 16, 16)] = zeros16
        return _

    lax.fori_loop(0, CHUNK * 8, zbuf_body, 0)
    for i in range(RPT // CHUNK):
        pltpu.sync_copy(zbuf_v, acc_sh.at[pl.ds(sid * RPT + i * CHUNK, CHUNK)])
    plsc.subcore_barrier()

    def edge_body(j, _):
        pltpu.async_copy(g_hbm.at[src_v.at[j]], rows_v, gsem).wait()
        pltpu.sync_copy(rows_v, acc_sh.at[dst_v.at[j]], add=True)
        return _

    lax.fori_loop(0, NCH, edge_body, 0)
    plsc.subcore_barrier()
    pltpu.sync_copy(acc_sh.at[pl.ds(sid * ORPT, ORPT)],
                    out_hbm.at[cid].at[pl.ds(sid * ORPT, ORPT)])


@functools.partial(
    pl.kernel,
    out_type=jax.ShapeDtypeStruct((NW, NB), jnp.float32),
    mesh=_mesh,
    scratch_types=[
        pltpu.VMEM((NCH, CHUNK), jnp.int32),
        pltpu.VMEM((NCH, CHUNK), jnp.int32),
        pltpu.VMEM((NB,), jnp.float32),
        pltpu.VMEM((NB,), jnp.float32),
    ],
)
def _scalagg_kernel(src_hbm, dst_hbm, g2_hbm, out_hbm,
                    src_v, dst_v, tab_v, acc_v):
    wid = lax.axis_index("c") * NS + lax.axis_index("s")
    pltpu.sync_copy(src_hbm.at[wid], src_v)
    pltpu.sync_copy(dst_hbm.at[wid], dst_v)
    pltpu.sync_copy(g2_hbm, tab_v)
    zeros16 = jnp.zeros((16,), jnp.float32)

    def zero_body(i, _):
        acc_v[pl.ds(i * 16, 16)] = zeros16
        return _

    lax.fori_loop(0, NB // 16, zero_body, 0)

    def edge_body(j, _):
        for k in range(CHUNK // 16):
            s = src_v[j, pl.ds(k * 16, 16)]
            d = dst_v[j, pl.ds(k * 16, 16)]
            vals = plsc.load_gather(tab_v, [s])
            plsc.addupdate_scatter(acc_v, [d], vals)
        return _

    lax.fori_loop(0, NCH, edge_body, 0)
    pltpu.sync_copy(acc_v, out_hbm.at[wid])


def _mm_scale_body(x_ref, w_ref, degp_ref, g_ref):
    deg = jnp.sum(degp_ref[...], axis=0) + 1.0
    dinv = lax.rsqrt(deg)
    h = jnp.dot(x_ref[...], w_ref[...], preferred_element_type=jnp.float32)
    g_ref[...] = h * dinv[:, None]


def _mm_scale(x, W1, degp):
    R = 1000
    return pl.pallas_call(
        _mm_scale_body,
        grid=(N // R,),
        in_specs=[
            pl.BlockSpec((R, D), lambda i: (i, 0)),
            pl.BlockSpec((D, D), lambda i: (0, 0)),
            pl.BlockSpec((NW, R), lambda i: (0, i)),
        ],
        out_specs=pl.BlockSpec((R, D), lambda i: (i, 0)),
        out_shape=jax.ShapeDtypeStruct((N, D), jnp.float32),
    )(x, W1, degp)


def _layer2_body(p_ref, g_ref, degp_ref, b1_ref, w2_ref, g2_ref):
    deg = jnp.sum(degp_ref[...], axis=0) + 1.0
    dinv = lax.rsqrt(deg)
    acc = p_ref[0] + p_ref[1] + g_ref[...]
    out1 = jnp.maximum(acc * dinv[:, None] + b1_ref[...], 0.0)
    h2 = jnp.dot(out1, w2_ref[...], preferred_element_type=jnp.float32)
    g2_ref[...] = h2 * dinv[:, None]


def _layer2(P, g, degp, b1, W2):
    R = 1000
    return pl.pallas_call(
        _layer2_body,
        grid=(N // R,),
        in_specs=[
            pl.BlockSpec((NC, R, D), lambda i: (0, i, 0)),
            pl.BlockSpec((R, D), lambda i: (i, 0)),
            pl.BlockSpec((NW, R), lambda i: (0, i)),
            pl.BlockSpec((1, D), lambda i: (0, 0)),
            pl.BlockSpec((D, 1), lambda i: (0, 0)),
        ],
        out_specs=pl.BlockSpec((R, 1), lambda i: (i, 0)),
        out_shape=jax.ShapeDtypeStruct((N, 1), jnp.float32),
    )(P, g, degp, b1, W2)


def _final_body(accp_ref, g2_ref, degp_ref, b2_ref, y_ref):
    deg = jnp.sum(degp_ref[...], axis=0) + 1.0
    dinv = lax.rsqrt(deg)
    val = (jnp.sum(accp_ref[...], axis=0) + g2_ref[..., 0]) * dinv + b2_ref[0, 0]
    y_ref[...] = (1.0 / (1.0 + jnp.exp(-val)))[:, None]


def _final(accp, g2, degp, b2):
    R = 1000
    return pl.pallas_call(
        _final_body,
        grid=(N // R,),
        in_specs=[
            pl.BlockSpec((NW, R), lambda i: (0, i)),
            pl.BlockSpec((R, 1), lambda i: (i, 0)),
            pl.BlockSpec((NW, R), lambda i: (0, i)),
            pl.BlockSpec((1, 1), lambda i: (0, 0)),
        ],
        out_specs=pl.BlockSpec((R, 1), lambda i: (i, 0)),
        out_shape=jax.ShapeDtypeStruct((N, 1), jnp.float32),
    )(accp, g2, degp, b2)


def kernel(x, edge_index, W1, b1, W2, b2):
    src = edge_index[0].astype(jnp.int32)
    dst = edge_index[1].astype(jnp.int32)
    pad = EPAD - E
    src_p = jnp.concatenate(
        [src, jnp.zeros((pad,), jnp.int32)]).reshape(NW, NCH, CHUNK)
    dst_p = jnp.concatenate(
        [dst, jnp.full((pad,), N, jnp.int32)]).reshape(NW, NCH, CHUNK)

    degp = _deg_kernel(dst_p)
    degp = degp[:, :N]
    g = _mm_scale(x, W1, degp)
    P = _rowagg_kernel(src_p, dst_p, g)
    g2 = _layer2(P, g, degp, b1.reshape(1, D), W2)
    g2t = jnp.pad(g2[:, 0], (0, NB - N))
    accp = _scalagg_kernel(src_p, dst_p, g2t)
    y = _final(accp[:, :N], g2, degp, b2.reshape(1, 1))
    return y


# speedup vs baseline: 14.9118x; 14.9118x over previous
import functools

import jax
import jax.numpy as jnp
from jax import lax
from jax.experimental import pallas as pl
from jax.experimental.pallas import tpu as pltpu
from jax.experimental.pallas import tpu_sc as plsc

N = 10000
E = 320000
D = 128
NC = 2
NS = 16
NW = NC * NS
CHUNK = 128
NCH = 80
EPW = NCH * CHUNK
EPAD = NW * EPW
NB = 10240
RPT = NB // NS
ORPT = N // NS

_mesh = plsc.VectorSubcoreMesh(core_axis_name="c", subcore_axis_name="s")
_sc_params = pltpu.CompilerParams(needs_layout_passes=False)


@functools.partial(
    pl.kernel,
    out_type=jax.ShapeDtypeStruct((NW, NB), jnp.float32),
    mesh=_mesh,
    scratch_types=[
        pltpu.VMEM((NCH, CHUNK), jnp.int32),
        pltpu.VMEM((NB,), jnp.float32),
    ],
    compiler_params=_sc_params,
)
def _deg_kernel(dst_hbm, out_hbm, dst_v, hist_v):
    wid = lax.axis_index("c") * NS + lax.axis_index("s")
    pltpu.sync_copy(dst_hbm.at[wid], dst_v)
    zeros16 = jnp.zeros((16,), jnp.float32)

    def zero_body(i, _):
        hist_v[pl.ds(i * 16, 16)] = zeros16
        return _

    lax.fori_loop(0, NB // 16, zero_body, 0)
    ones16 = jnp.ones((16,), jnp.float32)

    def edge_body(j, _):
        for k in range(CHUNK // 16):
            idx = dst_v[j, pl.ds(k * 16, 16)]
            plsc.addupdate_scatter(hist_v, [idx], ones16)
        return _

    lax.fori_loop(0, NCH, edge_body, 0)
    pltpu.sync_copy(hist_v, out_hbm.at[wid])


@functools.partial(
    pl.kernel,
    out_type=jax.ShapeDtypeStruct((NC, NB, D), jnp.float32),
    mesh=_mesh,
    scratch_types=[
        pltpu.VMEM((NCH, CHUNK), jnp.int32),
        pltpu.VMEM((NCH, CHUNK), jnp.int32),
        pltpu.VMEM((CHUNK, D), jnp.float32),
        pltpu.VMEM_SHARED((NB, D), jnp.float32),
        pltpu.SemaphoreType.DMA,
    ],
    compiler_params=_sc_params,
)
def _rowagg_kernel(src_hbm, dst_hbm, g_hbm, out_hbm,
                   src_v, dst_v, rows_v, acc_sh, gsem):
    cid = lax.axis_index("c")
    sid = lax.axis_index("s")
    wid = cid * NS + sid
    pltpu.sync_copy(src_hbm.at[wid], src_v)
    pltpu.sync_copy(dst_hbm.at[wid], dst_v)

    zeros16 = jnp.zeros((16,), jnp.float32)

    def zbuf_body(i, _):
        rows_v[i // 8, pl.ds((i % 8) * 16, 16)] = zeros16
        return _

    lax.fori_loop(0, CHUNK * 8, zbuf_body, 0)
    for i in range(RPT // CHUNK):
        pltpu.sync_copy(rows_v, acc_sh.at[pl.ds(sid * RPT + i * CHUNK, CHUNK)])
    plsc.subcore_barrier()

    def edge_body(j, _):
        pltpu.async_copy(g_hbm.at[src_v.at[j]], rows_v, gsem).wait()
        pltpu.sync_copy(rows_v, acc_sh.at[dst_v.at[j]], add=True)
        return _

    lax.fori_loop(0, NCH, edge_body, 0)
    plsc.subcore_barrier()
    pltpu.sync_copy(acc_sh.at[pl.ds(sid * RPT, RPT)],
                    out_hbm.at[cid].at[pl.ds(sid * RPT, RPT)])


@functools.partial(
    pl.kernel,
    out_type=jax.ShapeDtypeStruct((NW, NB), jnp.float32),
    mesh=_mesh,
    scratch_types=[
        pltpu.VMEM((NCH, CHUNK), jnp.int32),
        pltpu.VMEM((NCH, CHUNK), jnp.int32),
        pltpu.VMEM((NB,), jnp.float32),
        pltpu.VMEM((NB,), jnp.float32),
    ],
    compiler_params=_sc_params,
)
def _scalagg_kernel(src_hbm, dst_hbm, g2_hbm, out_hbm,
                    src_v, dst_v, tab_v, acc_v):
    wid = lax.axis_index("c") * NS + lax.axis_index("s")
    pltpu.sync_copy(src_hbm.at[wid], src_v)
    pltpu.sync_copy(dst_hbm.at[wid], dst_v)
    pltpu.sync_copy(g2_hbm, tab_v)
    zeros16 = jnp.zeros((16,), jnp.float32)

    def zero_body(i, _):
        acc_v[pl.ds(i * 16, 16)] = zeros16
        return _

    lax.fori_loop(0, NB // 16, zero_body, 0)

    def edge_body(j, _):
        for k in range(CHUNK // 16):
            s = src_v[j, pl.ds(k * 16, 16)]
            d = dst_v[j, pl.ds(k * 16, 16)]
            vals = plsc.load_gather(tab_v, [s])
            plsc.addupdate_scatter(acc_v, [d], vals)
        return _

    lax.fori_loop(0, NCH, edge_body, 0)
    pltpu.sync_copy(acc_v, out_hbm.at[wid])


def _mm_scale_body(x_ref, w_ref, degp_ref, g_ref):
    deg = jnp.sum(degp_ref[...], axis=1) + 1.0
    dinv = lax.rsqrt(deg)
    h = jnp.dot(x_ref[...], w_ref[...], preferred_element_type=jnp.float32)
    g_ref[...] = h * dinv[:, None]


def _mm_scale(x, W1, degp):
    R = 1000
    return pl.pallas_call(
        _mm_scale_body,
        grid=(N // R,),
        in_specs=[
            pl.BlockSpec((R, D), lambda i: (i, 0)),
            pl.BlockSpec((D, D), lambda i: (0, 0)),
            pl.BlockSpec((R, NW), lambda i: (i, 0)),
        ],
        out_specs=pl.BlockSpec((R, D), lambda i: (i, 0)),
        out_shape=jax.ShapeDtypeStruct((N, D), jnp.float32),
    )(x, W1, degp)


def _layer2_body(p_ref, g_ref, degp_ref, b1_ref, w2_ref, g2_ref):
    deg = jnp.sum(degp_ref[...], axis=1) + 1.0
    dinv = lax.rsqrt(deg)
    acc = p_ref[0] + p_ref[1] + g_ref[...]
    out1 = jnp.maximum(acc * dinv[:, None] + b1_ref[...], 0.0)
    h2 = jnp.dot(out1, w2_ref[...], preferred_element_type=jnp.float32)
    g2_ref[...] = h2 * dinv[:, None]


def _layer2(P, g, degp, b1, W2):
    R = 1000
    return pl.pallas_call(
        _layer2_body,
        grid=(N // R,),
        in_specs=[
            pl.BlockSpec((NC, R, D), lambda i: (0, i, 0)),
            pl.BlockSpec((R, D), lambda i: (i, 0)),
            pl.BlockSpec((R, NW), lambda i: (i, 0)),
            pl.BlockSpec((1, D), lambda i: (0, 0)),
            pl.BlockSpec((D, 1), lambda i: (0, 0)),
        ],
        out_specs=pl.BlockSpec((R, 1), lambda i: (i, 0)),
        out_shape=jax.ShapeDtypeStruct((N, 1), jnp.float32),
    )(P, g, degp, b1, W2)


def _final_body(accp_ref, g2_ref, degp_ref, b2_ref, y_ref):
    deg = jnp.sum(degp_ref[...], axis=1) + 1.0
    dinv = lax.rsqrt(deg)
    val = (jnp.sum(accp_ref[...], axis=1) + g2_ref[..., 0]) * dinv + b2_ref[0, 0]
    y_ref[...] = (1.0 / (1.0 + jnp.exp(-val)))[:, None]


def _final(accp, g2, degp, b2):
    R = 1000
    return pl.pallas_call(
        _final_body,
        grid=(N // R,),
        in_specs=[
            pl.BlockSpec((R, NW), lambda i: (i, 0)),
            pl.BlockSpec((R, 1), lambda i: (i, 0)),
            pl.BlockSpec((R, NW), lambda i: (i, 0)),
            pl.BlockSpec((1, 1), lambda i: (0, 0)),
        ],
        out_specs=pl.BlockSpec((R, 1), lambda i: (i, 0)),
        out_shape=jax.ShapeDtypeStruct((N, 1), jnp.float32),
    )(accp, g2, degp, b2)


def kernel(x, edge_index, W1, b1, W2, b2):
    src = edge_index[0].astype(jnp.int32)
    dst = edge_index[1].astype(jnp.int32)
    pad = EPAD - E
    src_p = jnp.concatenate(
        [src, jnp.zeros((pad,), jnp.int32)]).reshape(NW, NCH, CHUNK)
    dst_p = jnp.concatenate(
        [dst, jnp.full((pad,), N, jnp.int32)]).reshape(NW, NCH, CHUNK)

    degp = _deg_kernel(dst_p)
    degp = degp[:, :N].T
    g = _mm_scale(x, W1, degp)
    P = _rowagg_kernel(src_p, dst_p, g)
    g2 = _layer2(P, g, degp, b1.reshape(1, D), W2)
    g2t = jnp.pad(g2[:, 0], (0, NB - N))
    accp = _scalagg_kernel(src_p, dst_p, g2t)
    y = _final(accp[:, :N].T, g2, degp, b2.reshape(1, 1))
    return y


# speedup vs baseline: 15.9697x; 1.0709x over previous
import functools

import jax
import jax.numpy as jnp
from jax import lax
from jax.experimental import pallas as pl
from jax.experimental.pallas import tpu as pltpu
from jax.experimental.pallas import tpu_sc as plsc

N = 10000
E = 320000
D = 128
NC = 2
NS = 16
NW = NC * NS
CHUNK = 128
NCH = 80
EPW = NCH * CHUNK
EPAD = NW * EPW
NB = 10240
RPT = NB // NS
ORPT = N // NS

_mesh = plsc.VectorSubcoreMesh(core_axis_name="c", subcore_axis_name="s")
_sc_params = pltpu.CompilerParams(needs_layout_passes=False)


@functools.partial(
    pl.kernel,
    out_type=jax.ShapeDtypeStruct((NW, NB), jnp.float32),
    mesh=_mesh,
    scratch_types=[
        pltpu.VMEM((NCH, CHUNK), jnp.int32),
        pltpu.VMEM((NB,), jnp.float32),
    ],
    compiler_params=_sc_params,
)
def _deg_kernel(dst_hbm, out_hbm, dst_v, hist_v):
    wid = lax.axis_index("c") * NS + lax.axis_index("s")
    pltpu.sync_copy(dst_hbm.at[wid], dst_v)
    zeros16 = jnp.zeros((16,), jnp.float32)

    def zero_body(i, _):
        hist_v[pl.ds(i * 16, 16)] = zeros16
        return _

    lax.fori_loop(0, NB // 16, zero_body, 0)
    ones16 = jnp.ones((16,), jnp.float32)

    def edge_body(j, _):
        for k in range(CHUNK // 16):
            idx = dst_v[j, pl.ds(k * 16, 16)]
            plsc.addupdate_scatter(hist_v, [idx], ones16)
        return _

    lax.fori_loop(0, NCH, edge_body, 0)
    pltpu.sync_copy(hist_v, out_hbm.at[wid])


SEG = 16
NSEG = NCH // SEG


@functools.partial(
    pl.kernel,
    out_type=jax.ShapeDtypeStruct((NC, NB, D), jnp.float32),
    mesh=_mesh,
    scratch_types=[
        pltpu.VMEM((SEG, CHUNK), jnp.int32),
        pltpu.VMEM((SEG, CHUNK), jnp.int32),
        pltpu.VMEM((2, CHUNK, D), jnp.float32),
        pltpu.VMEM_SHARED((NB, D), jnp.float32),
        pltpu.SemaphoreType.DMA,
        pltpu.SemaphoreType.DMA,
    ],
    compiler_params=_sc_params,
)
def _rowagg_kernel(src_hbm, dst_hbm, g_hbm, out_hbm,
                   src_v, dst_v, rows_v, acc_sh, gsem0, gsem1):
    cid = lax.axis_index("c")
    sid = lax.axis_index("s")
    wid = cid * NS + sid
    gsems = (gsem0, gsem1)

    zeros16 = jnp.zeros((16,), jnp.float32)

    def zbuf_body(i, _):
        rows_v[0, i // 8, pl.ds((i % 8) * 16, 16)] = zeros16
        return _

    lax.fori_loop(0, CHUNK * 8, zbuf_body, 0)
    for i in range(RPT // CHUNK):
        pltpu.sync_copy(rows_v.at[0],
                        acc_sh.at[pl.ds(sid * RPT + i * CHUNK, CHUNK)])
    plsc.subcore_barrier()

    for seg in range(NSEG):
        pltpu.sync_copy(src_hbm.at[wid].at[pl.ds(seg * SEG, SEG)], src_v)
        pltpu.sync_copy(dst_hbm.at[wid].at[pl.ds(seg * SEG, SEG)], dst_v)
        for b in range(2):
            pltpu.async_copy(g_hbm.at[src_v.at[b]], rows_v.at[b], gsems[b])

        def pair_body(p, carry):
            for b in range(2):
                t = 2 * p + b
                pltpu.make_async_copy(
                    g_hbm.at[src_v.at[0]], rows_v.at[b], gsems[b]).wait()
                pltpu.sync_copy(rows_v.at[b], acc_sh.at[dst_v.at[t]],
                                add=True)

                @pl.when(t + 2 < SEG)
                def _start_next(b=b, t=t):
                    pltpu.async_copy(g_hbm.at[src_v.at[t + 2]],
                                     rows_v.at[b], gsems[b])
            return carry

        lax.fori_loop(0, SEG // 2, pair_body, 0)
    plsc.subcore_barrier()
    pltpu.sync_copy(acc_sh.at[pl.ds(sid * RPT, RPT)],
                    out_hbm.at[cid].at[pl.ds(sid * RPT, RPT)])


@functools.partial(
    pl.kernel,
    out_type=jax.ShapeDtypeStruct((NW, NB), jnp.float32),
    mesh=_mesh,
    scratch_types=[
        pltpu.VMEM((NCH, CHUNK), jnp.int32),
        pltpu.VMEM((NCH, CHUNK), jnp.int32),
        pltpu.VMEM((NB,), jnp.float32),
        pltpu.VMEM((NB,), jnp.float32),
    ],
    compiler_params=_sc_params,
)
def _scalagg_kernel(src_hbm, dst_hbm, g2_hbm, out_hbm,
                    src_v, dst_v, tab_v, acc_v):
    wid = lax.axis_index("c") * NS + lax.axis_index("s")
    pltpu.sync_copy(src_hbm.at[wid], src_v)
    pltpu.sync_copy(dst_hbm.at[wid], dst_v)
    pltpu.sync_copy(g2_hbm, tab_v)
    zeros16 = jnp.zeros((16,), jnp.float32)

    def zero_body(i, _):
        acc_v[pl.ds(i * 16, 16)] = zeros16
        return _

    lax.fori_loop(0, NB // 16, zero_body, 0)

    def edge_body(j, _):
        for k in range(CHUNK // 16):
            s = src_v[j, pl.ds(k * 16, 16)]
            d = dst_v[j, pl.ds(k * 16, 16)]
            vals = plsc.load_gather(tab_v, [s])
            plsc.addupdate_scatter(acc_v, [d], vals)
        return _

    lax.fori_loop(0, NCH, edge_body, 0)
    pltpu.sync_copy(acc_v, out_hbm.at[wid])


def _mm_scale_body(x_ref, w_ref, degp_ref, g_ref):
    deg = jnp.sum(degp_ref[...], axis=1) + 1.0
    dinv = lax.rsqrt(deg)
    h = jnp.dot(x_ref[...], w_ref[...], preferred_element_type=jnp.float32)
    g_ref[...] = h * dinv[:, None]


def _mm_scale(x, W1, degp):
    R = 1000
    return pl.pallas_call(
        _mm_scale_body,
        grid=(N // R,),
        in_specs=[
            pl.BlockSpec((R, D), lambda i: (i, 0)),
            pl.BlockSpec((D, D), lambda i: (0, 0)),
            pl.BlockSpec((R, NW), lambda i: (i, 0)),
        ],
        out_specs=pl.BlockSpec((R, D), lambda i: (i, 0)),
        out_shape=jax.ShapeDtypeStruct((N, D), jnp.float32),
    )(x, W1, degp)


def _layer2_body(p_ref, g_ref, degp_ref, b1_ref, w2_ref, g2_ref):
    deg = jnp.sum(degp_ref[...], axis=1) + 1.0
    dinv = lax.rsqrt(deg)
    acc = p_ref[0] + p_ref[1] + g_ref[...]
    out1 = jnp.maximum(acc * dinv[:, None] + b1_ref[...], 0.0)
    h2 = jnp.dot(out1, w2_ref[...], preferred_element_type=jnp.float32)
    g2_ref[...] = h2 * dinv[:, None]


def _layer2(P, g, degp, b1, W2):
    R = 1000
    return pl.pallas_call(
        _layer2_body,
        grid=(N // R,),
        in_specs=[
            pl.BlockSpec((NC, R, D), lambda i: (0, i, 0)),
            pl.BlockSpec((R, D), lambda i: (i, 0)),
            pl.BlockSpec((R, NW), lambda i: (i, 0)),
            pl.BlockSpec((1, D), lambda i: (0, 0)),
            pl.BlockSpec((D, 1), lambda i: (0, 0)),
        ],
        out_specs=pl.BlockSpec((R, 1), lambda i: (i, 0)),
        out_shape=jax.ShapeDtypeStruct((N, 1), jnp.float32),
    )(P, g, degp, b1, W2)


def _final_body(accp_ref, g2_ref, degp_ref, b2_ref, y_ref):
    deg = jnp.sum(degp_ref[...], axis=1) + 1.0
    dinv = lax.rsqrt(deg)
    val = (jnp.sum(accp_ref[...], axis=1) + g2_ref[..., 0]) * dinv + b2_ref[0, 0]
    y_ref[...] = (1.0 / (1.0 + jnp.exp(-val)))[:, None]


def _final(accp, g2, degp, b2):
    R = 1000
    return pl.pallas_call(
        _final_body,
        grid=(N // R,),
        in_specs=[
            pl.BlockSpec((R, NW), lambda i: (i, 0)),
            pl.BlockSpec((R, 1), lambda i: (i, 0)),
            pl.BlockSpec((R, NW), lambda i: (i, 0)),
            pl.BlockSpec((1, 1), lambda i: (0, 0)),
        ],
        out_specs=pl.BlockSpec((R, 1), lambda i: (i, 0)),
        out_shape=jax.ShapeDtypeStruct((N, 1), jnp.float32),
    )(accp, g2, degp, b2)


def kernel(x, edge_index, W1, b1, W2, b2):
    src = edge_index[0].astype(jnp.int32)
    dst = edge_index[1].astype(jnp.int32)
    pad = EPAD - E
    src_p = jnp.concatenate(
        [src, jnp.zeros((pad,), jnp.int32)]).reshape(NW, NCH, CHUNK)
    dst_p = jnp.concatenate(
        [dst, jnp.full((pad,), N, jnp.int32)]).reshape(NW, NCH, CHUNK)

    degp = _deg_kernel(dst_p)
    degp = degp[:, :N].T
    g = _mm_scale(x, W1, degp)
    P = _rowagg_kernel(src_p, dst_p, g)
    g2 = _layer2(P, g, degp, b1.reshape(1, D), W2)
    g2t = jnp.pad(g2[:, 0], (0, NB - N))
    accp = _scalagg_kernel(src_p, dst_p, g2t)
    y = _final(accp[:, :N].T, g2, degp, b2.reshape(1, 1))
    return y


# speedup vs baseline: 23.8548x; 1.4938x over previous
import functools

import jax
import jax.numpy as jnp
from jax import lax
from jax.experimental import pallas as pl
from jax.experimental.pallas import tpu as pltpu
from jax.experimental.pallas import tpu_sc as plsc

N = 10000
E = 320000
D = 128
NC = 2
NS = 16
NW = NC * NS
CHUNK = 128
NCH = 80
EPW = NCH * CHUNK
EPAD = NW * EPW
NB = 10240
RPT = NB // NS
ORPT = N // NS

_mesh = plsc.VectorSubcoreMesh(core_axis_name="c", subcore_axis_name="s")
_sc_params = pltpu.CompilerParams(needs_layout_passes=False)
_sc_params_notc = pltpu.CompilerParams(needs_layout_passes=False,
                                       use_tc_tiling_on_sc=False)


@functools.partial(
    pl.kernel,
    out_type=jax.ShapeDtypeStruct((NW, NB), jnp.float32),
    mesh=_mesh,
    scratch_types=[
        pltpu.VMEM((NCH, CHUNK), jnp.int32),
        pltpu.VMEM((NB,), jnp.float32),
    ],
    compiler_params=_sc_params,
)
def _deg_kernel(dst_hbm, out_hbm, dst_v, hist_v):
    wid = lax.axis_index("c") * NS + lax.axis_index("s")
    pltpu.sync_copy(dst_hbm.at[wid], dst_v)
    zeros16 = jnp.zeros((16,), jnp.float32)

    def zero_body(i, _):
        hist_v[pl.ds(i * 16, 16)] = zeros16
        return _

    lax.fori_loop(0, NB // 16, zero_body, 0)
    ones16 = jnp.ones((16,), jnp.float32)

    def edge_body(j, _):
        for k in range(CHUNK // 16):
            idx = dst_v[j, pl.ds(k * 16, 16)]
            plsc.addupdate_scatter(hist_v, [idx], ones16)
        return _

    lax.fori_loop(0, NCH, edge_body, 0)
    pltpu.sync_copy(hist_v, out_hbm.at[wid])


DH = D // NC
NCH2 = EPAD // (NS * CHUNK)
NBUF = 4
RPT2 = NB // NS


@functools.partial(
    pl.kernel,
    out_type=(jax.ShapeDtypeStruct((NB, DH), jnp.float32),
              jax.ShapeDtypeStruct((NB, DH), jnp.float32)),
    mesh=_mesh,
    scratch_types=[
        pltpu.VMEM((NCH2, CHUNK), jnp.int32),
        pltpu.VMEM((NCH2, CHUNK), jnp.int32),
        pltpu.VMEM((NBUF, CHUNK, DH), jnp.float32),
        pltpu.VMEM_SHARED((NB, DH), jnp.float32),
        pltpu.SemaphoreType.DMA,
        pltpu.SemaphoreType.DMA,
        pltpu.SemaphoreType.DMA,
        pltpu.SemaphoreType.DMA,
    ],
    compiler_params=_sc_params_notc,
)
def _rowagg_kernel(src_hbm, dst_hbm, g0_hbm, g1_hbm, out0_hbm, out1_hbm,
                   src_v, dst_v, rows_v, acc_sh, gsem0, gsem1, gsem2, gsem3):
    cid = lax.axis_index("c")
    sid = lax.axis_index("s")
    gsems = (gsem0, gsem1, gsem2, gsem3)

    pltpu.sync_copy(src_hbm.at[sid], src_v)
    pltpu.sync_copy(dst_hbm.at[sid], dst_v)

    zeros16 = jnp.zeros((16,), jnp.float32)

    def zbuf_body(i, _):
        rows_v[0, i // (DH // 16), pl.ds((i % (DH // 16)) * 16, 16)] = zeros16
        return _

    lax.fori_loop(0, CHUNK * (DH // 16), zbuf_body, 0)
    for i in range(RPT2 // CHUNK):
        pltpu.sync_copy(rows_v.at[0],
                        acc_sh.at[pl.ds(sid * RPT2 + i * CHUNK, CHUNK)])
    plsc.subcore_barrier()

    def gstart(t, b):
        @pl.when(cid == 0)
        def _g0():
            pltpu.async_copy(g0_hbm.at[src_v.at[t]], rows_v.at[b], gsems[b])

        @pl.when(cid == 1)
        def _g1():
            pltpu.async_copy(g1_hbm.at[src_v.at[t]], rows_v.at[b], gsems[b])

    for b in range(NBUF):
        gstart(b, b)

    def quad_body(p, carry):
        for b in range(NBUF):
            t = NBUF * p + b
            pltpu.make_async_copy(
                g0_hbm.at[src_v.at[0]], rows_v.at[b], gsems[b]).wait()
            pltpu.sync_copy(rows_v.at[b], acc_sh.at[dst_v.at[t]], add=True)

            @pl.when(t + NBUF < NCH2)
            def _next(b=b, t=t):
                gstart(t + NBUF, b)
        return carry

    lax.fori_loop(0, NCH2 // NBUF, quad_body, 0)
    plsc.subcore_barrier()

    @pl.when(cid == 0)
    def _store0():
        pltpu.sync_copy(acc_sh.at[pl.ds(sid * RPT2, RPT2)],
                        out0_hbm.at[pl.ds(sid * RPT2, RPT2)])

    @pl.when(cid == 1)
    def _store1():
        pltpu.sync_copy(acc_sh.at[pl.ds(sid * RPT2, RPT2)],
                        out1_hbm.at[pl.ds(sid * RPT2, RPT2)])


@functools.partial(
    pl.kernel,
    out_type=jax.ShapeDtypeStruct((NW, NB), jnp.float32),
    mesh=_mesh,
    scratch_types=[
        pltpu.VMEM((NCH, CHUNK), jnp.int32),
        pltpu.VMEM((NCH, CHUNK), jnp.int32),
        pltpu.VMEM((NB,), jnp.float32),
        pltpu.VMEM((NB,), jnp.float32),
    ],
    compiler_params=_sc_params,
)
def _scalagg_kernel(src_hbm, dst_hbm, g2_hbm, out_hbm,
                    src_v, dst_v, tab_v, acc_v):
    wid = lax.axis_index("c") * NS + lax.axis_index("s")
    pltpu.sync_copy(src_hbm.at[wid], src_v)
    pltpu.sync_copy(dst_hbm.at[wid], dst_v)
    pltpu.sync_copy(g2_hbm, tab_v)
    zeros16 = jnp.zeros((16,), jnp.float32)

    def zero_body(i, _):
        acc_v[pl.ds(i * 16, 16)] = zeros16
        return _

    lax.fori_loop(0, NB // 16, zero_body, 0)

    def edge_body(j, _):
        for k in range(CHUNK // 16):
            s = src_v[j, pl.ds(k * 16, 16)]
            d = dst_v[j, pl.ds(k * 16, 16)]
            vals = plsc.load_gather(tab_v, [s])
            plsc.addupdate_scatter(acc_v, [d], vals)
        return _

    lax.fori_loop(0, NCH, edge_body, 0)
    pltpu.sync_copy(acc_v, out_hbm.at[wid])


def _mm_scale_body(x_ref, w_ref, degp_ref, g0_ref, g1_ref):
    deg = jnp.sum(degp_ref[...], axis=1) + 1.0
    dinv = lax.rsqrt(deg)
    h = jnp.dot(x_ref[...], w_ref[...], preferred_element_type=jnp.float32)
    hd = h * dinv[:, None]
    g0_ref[...] = hd[:, :DH]
    g1_ref[...] = hd[:, DH:]


def _mm_scale(x, W1, degp):
    R = 1000
    return pl.pallas_call(
        _mm_scale_body,
        grid=(N // R,),
        in_specs=[
            pl.BlockSpec((R, D), lambda i: (i, 0)),
            pl.BlockSpec((D, D), lambda i: (0, 0)),
            pl.BlockSpec((R, NW), lambda i: (i, 0)),
        ],
        out_specs=[
            pl.BlockSpec((R, DH), lambda i: (i, 0)),
            pl.BlockSpec((R, DH), lambda i: (i, 0)),
        ],
        out_shape=[
            jax.ShapeDtypeStruct((NB, DH), jnp.float32),
            jax.ShapeDtypeStruct((NB, DH), jnp.float32),
        ],
    )(x, W1, degp)


def _layer2_body(p0_ref, p1_ref, g0_ref, g1_ref, degp_ref, b1_ref, w2_ref,
                 g2_ref):
    deg = jnp.sum(degp_ref[...], axis=1) + 1.0
    dinv = lax.rsqrt(deg)
    acc = jnp.concatenate(
        [p0_ref[...] + g0_ref[...], p1_ref[...] + g1_ref[...]], axis=1)
    out1 = jnp.maximum(acc * dinv[:, None] + b1_ref[...], 0.0)
    h2 = jnp.dot(out1, w2_ref[...], preferred_element_type=jnp.float32)
    g2_ref[...] = h2 * dinv[:, None]


def _layer2(P0, P1, g0, g1, degp, b1, W2):
    R = 1000
    half = pl.BlockSpec((R, DH), lambda i: (i, 0))
    return pl.pallas_call(
        _layer2_body,
        grid=(N // R,),
        in_specs=[
            half, half, half, half,
            pl.BlockSpec((R, NW), lambda i: (i, 0)),
            pl.BlockSpec((1, D), lambda i: (0, 0)),
            pl.BlockSpec((D, 1), lambda i: (0, 0)),
        ],
        out_specs=pl.BlockSpec((R, 1), lambda i: (i, 0)),
        out_shape=jax.ShapeDtypeStruct((N, 1), jnp.float32),
    )(P0, P1, g0, g1, degp, b1, W2)


def _final_body(accp_ref, g2_ref, degp_ref, b2_ref, y_ref):
    deg = jnp.sum(degp_ref[...], axis=1) + 1.0
    dinv = lax.rsqrt(deg)
    val = (jnp.sum(accp_ref[...], axis=1) + g2_ref[..., 0]) * dinv + b2_ref[0, 0]
    y_ref[...] = (1.0 / (1.0 + jnp.exp(-val)))[:, None]


def _final(accp, g2, degp, b2):
    R = 1000
    return pl.pallas_call(
        _final_body,
        grid=(N // R,),
        in_specs=[
            pl.BlockSpec((R, NW), lambda i: (i, 0)),
            pl.BlockSpec((R, 1), lambda i: (i, 0)),
            pl.BlockSpec((R, NW), lambda i: (i, 0)),
            pl.BlockSpec((1, 1), lambda i: (0, 0)),
        ],
        out_specs=pl.BlockSpec((R, 1), lambda i: (i, 0)),
        out_shape=jax.ShapeDtypeStruct((N, 1), jnp.float32),
    )(accp, g2, degp, b2)


def kernel(x, edge_index, W1, b1, W2, b2):
    src = edge_index[0].astype(jnp.int32)
    dst = edge_index[1].astype(jnp.int32)
    pad = EPAD - E
    src_flat = jnp.concatenate([src, jnp.zeros((pad,), jnp.int32)])
    dst_flat = jnp.concatenate([dst, jnp.full((pad,), N, jnp.int32)])
    src_p = src_flat.reshape(NW, NCH, CHUNK)
    dst_p = dst_flat.reshape(NW, NCH, CHUNK)
    src_t = src_flat.reshape(NS, NCH2, CHUNK)
    dst_t = dst_flat.reshape(NS, NCH2, CHUNK)

    degp = _deg_kernel(dst_p)
    degp = degp[:, :N].T
    g0, g1 = _mm_scale(x, W1, degp)
    P0, P1 = _rowagg_kernel(src_t, dst_t, g0, g1)
    g2 = _layer2(P0, P1, g0, g1, degp,
                 b1.reshape(1, D), W2)
    g2t = jnp.pad(g2[:, 0], (0, NB - N))
    accp = _scalagg_kernel(src_p, dst_p, g2t)
    y = _final(accp[:, :N].T, g2, degp, b2.reshape(1, 1))
    return y
